# baseline (speedup 1.0000x reference)
import functools

import numpy as np
import jax
import jax.numpy as jnp
from jax import lax
from jax.experimental import pallas as pl
from jax.experimental.pallas import tpu as pltpu
from jax.experimental.pallas import tpu_sc as plsc

_N = 10000
_NP = 10112
_E = 320000
_EP = 327680
_HID = 16
_HEADS = (8, 8, 8, 1)
_G = 64
_PREC = lax.Precision.HIGHEST

_BN = 2000
_BE = 2048
_WSC = 128
_NWORK = 32
_CHUNKS = _EP // (_NWORK * _WSC)
_ROWS = _NP // 16


def _hrep(nh, d):
    rows = lax.broadcasted_iota(jnp.int32, (nh, d), 0)
    cols = lax.broadcasted_iota(jnp.int32, (nh, d), 1)
    return (cols // (d // nh) == rows).astype(jnp.float32)


def _combine_prev(op_blk, dp_blk, s_blk):
    osum = op_blk[0] + op_blk[1]
    dsum = (dp_blk[0] + dp_blk[1])[:, :8]
    drep = jnp.dot(dsum, _hrep(8, 128), precision=_PREC)
    safe = jnp.where(drep > 0.0, drep, 1.0)
    return jax.nn.relu(jnp.where(drep > 0.0, osum / safe, 0.0) + s_blk)


def _weight_specs(din, dout):
    w = pl.BlockSpec((din, dout), lambda i: (0, 0))
    b = pl.BlockSpec((1, dout), lambda i: (0, 0))
    return [w, b, w, b, w, b, w, b]


def _proj(hb, w_ref, b_ref):
    return jnp.dot(hb, w_ref[...], precision=_PREC) + b_ref[...]


def _tc_qkvs_first(h, wq, bq, wk, bk, wv, bv, ws, bs):
    n, din = h.shape
    dout = wq.shape[1]

    def body(h_ref, wq_r, bq_r, wk_r, bk_r, wv_r, bv_r, ws_r, bs_r,
             q_o, k_o, v_o, s_o):
        hb = h_ref[...]
        q_o[...] = _proj(hb, wq_r, bq_r)
        k_o[...] = _proj(hb, wk_r, bk_r)
        v_o[...] = _proj(hb, wv_r, bv_r)
        s_o[...] = _proj(hb, ws_r, bs_r)

    return pl.pallas_call(
        body,
        grid=(n // _BN,),
        in_specs=[pl.BlockSpec((_BN, din), lambda i: (i, 0))] + _weight_specs(din, dout),
        out_specs=[pl.BlockSpec((_BN, dout), lambda i: (i, 0))] * 4,
        out_shape=[jax.ShapeDtypeStruct((n, dout), jnp.float32)] * 4,
    )(h, wq, bq, wk, bk, wv, bv, ws, bs)


def _tc_qkvs_next(op, dp, s_prev, wq, bq, wk, bk, wv, bv, ws, bs):
    n = s_prev.shape[0]
    din = s_prev.shape[1]
    dout = wq.shape[1]

    def body(op_ref, dp_ref, s_ref, wq_r, bq_r, wk_r, bk_r, wv_r, bv_r, ws_r, bs_r,
             q_o, k_o, v_o, s_o):
        hb = _combine_prev(op_ref[...], dp_ref[...], s_ref[...])
        q_o[...] = _proj(hb, wq_r, bq_r)
        k_o[...] = _proj(hb, wk_r, bk_r)
        v_o[...] = _proj(hb, wv_r, bv_r)
        s_o[...] = _proj(hb, ws_r, bs_r)

    return pl.pallas_call(
        body,
        grid=(n // _BN,),
        in_specs=[
            pl.BlockSpec((2, _BN, din), lambda i: (0, i, 0)),
            pl.BlockSpec((2, _BN, 128), lambda i: (0, i, 0)),
            pl.BlockSpec((_BN, din), lambda i: (i, 0)),
        ] + _weight_specs(din, dout),
        out_specs=[pl.BlockSpec((_BN, dout), lambda i: (i, 0))] * 4,
        out_shape=[jax.ShapeDtypeStruct((n, dout), jnp.float32)] * 4,
    )(op, dp, s_prev, wq, bq, wk, bk, wv, bv, ws, bs)


def _tc_qkvs_last(op, dp, s_prev, wq, bq, wk, bk, wv, bv, ws, bs):
    n = s_prev.shape[0]
    din = s_prev.shape[1]
    dout = wq.shape[1]

    def body(op_ref, dp_ref, s_ref, wq_r, bq_r, wk_r, bk_r, wv_r, bv_r, ws_r, bs_r,
             qkv_o, s_o):
        hb = _combine_prev(op_ref[...], dp_ref[...], s_ref[...])
        q = _proj(hb, wq_r, bq_r)
        k = _proj(hb, wk_r, bk_r)
        v = _proj(hb, wv_r, bv_r)
        z = jnp.zeros((q.shape[0], 128 - 3 * dout), jnp.float32)
        qkv_o[...] = jnp.concatenate([q, k, v, z], axis=1)
        s_o[...] = _proj(hb, ws_r, bs_r)

    return pl.pallas_call(
        body,
        grid=(n // _BN,),
        in_specs=[
            pl.BlockSpec((2, _BN, din), lambda i: (0, i, 0)),
            pl.BlockSpec((2, _BN, 128), lambda i: (0, i, 0)),
            pl.BlockSpec((_BN, din), lambda i: (i, 0)),
        ] + _weight_specs(din, dout),
        out_specs=[pl.BlockSpec((_BN, 128), lambda i: (i, 0)),
                   pl.BlockSpec((_BN, dout), lambda i: (i, 0))],
        out_shape=[jax.ShapeDtypeStruct((n, 128), jnp.float32),
                   jax.ShapeDtypeStruct((n, dout), jnp.float32)],
    )(op, dp, s_prev, wq, bq, wk, bk, wv, bv, ws, bs)


def _tc_alpha(qrows, krows, nh, qoff, koff):
    e = qrows.shape[0]
    dc = nh * _HID
    nblk = e // _BE

    def body(qd_ref, ks_ref, a_ref, m_ref):
        i = pl.program_id(0)
        prod = qd_ref[:, qoff:qoff + dc] * ks_ref[:, koff:koff + dc]
        rt = _hrep(nh, dc).T
        a = jnp.dot(prod, rt, precision=_PREC) * (1.0 / np.sqrt(_HID))
        if nh < 8:
            a = jnp.concatenate([a, jnp.zeros((a.shape[0], 8 - nh), jnp.float32)], axis=1)
        a_ref[...] = a
        bm = jnp.max(a, axis=0, keepdims=True)

        @pl.when(i == 0)
        def _():
            m_ref[...] = bm

        @pl.when(i > 0)
        def _():
            m_ref[...] = jnp.maximum(m_ref[...], bm)

    return pl.pallas_call(
        body,
        grid=(nblk,),
        in_specs=[pl.BlockSpec((_BE, 128), lambda i: (i, 0))] * 2,
        out_specs=[pl.BlockSpec((_BE, 8), lambda i: (i, 0)),
                   pl.BlockSpec((1, 8), lambda i: (0, 0))],
        out_shape=[jax.ShapeDtypeStruct((e, 8), jnp.float32),
                   jax.ShapeDtypeStruct((1, 8), jnp.float32)],
    )(qrows, krows)


def _tc_weight(alpha, gmax, vrows, nh, voff):
    e = alpha.shape[0]
    dc = nh * _HID
    nblk = e // _BE

    def body(a_ref, m_ref, vs_ref, ew_ref, vw_ref):
        i = pl.program_id(0)
        rows = lax.broadcasted_iota(jnp.int32, (_BE, 1), 0) + i * _BE
        valid = (rows < _E).astype(jnp.float32)
        e8 = jnp.exp(a_ref[...] - m_ref[...]) * valid
        ew_ref[...] = jnp.concatenate(
            [e8, jnp.zeros((e8.shape[0], 120), jnp.float32)], axis=1)
        vs = vs_ref[:, voff:voff + dc]
        vw_ref[...] = vs * jnp.dot(e8[:, :nh], _hrep(nh, dc), precision=_PREC)

    return pl.pallas_call(
        body,
        grid=(nblk,),
        in_specs=[pl.BlockSpec((_BE, 8), lambda i: (i, 0)),
                  pl.BlockSpec((1, 8), lambda i: (0, 0)),
                  pl.BlockSpec((_BE, 128), lambda i: (i, 0))],
        out_specs=[pl.BlockSpec((_BE, 128), lambda i: (i, 0)),
                   pl.BlockSpec((_BE, 128), lambda i: (i, 0))],
        out_shape=[jax.ShapeDtypeStruct((e, 128), jnp.float32),
                   jax.ShapeDtypeStruct((e, 128), jnp.float32)],
    )(alpha, gmax, vrows)


def _tc_weight1(alpha, gmax, vrows, voff):
    e = alpha.shape[0]
    nblk = e // _BE

    def body(a_ref, m_ref, vs_ref, cw_ref):
        i = pl.program_id(0)
        rows = lax.broadcasted_iota(jnp.int32, (_BE, 1), 0) + i * _BE
        valid = (rows < _E).astype(jnp.float32)
        e8 = jnp.exp(a_ref[...] - m_ref[...]) * valid
        vs = vs_ref[:, voff:voff + _HID]
        cw_ref[...] = jnp.concatenate(
            [vs * e8[:, 0:1], e8,
             jnp.zeros((e8.shape[0], 128 - _HID - 8), jnp.float32)], axis=1)

    return pl.pallas_call(
        body,
        grid=(nblk,),
        in_specs=[pl.BlockSpec((_BE, 8), lambda i: (i, 0)),
                  pl.BlockSpec((1, 8), lambda i: (0, 0)),
                  pl.BlockSpec((_BE, 128), lambda i: (i, 0))],
        out_specs=[pl.BlockSpec((_BE, 128), lambda i: (i, 0))],
        out_shape=[jax.ShapeDtypeStruct((e, 128), jnp.float32)],
    )(alpha, gmax, vrows)[0]


def _tc_pool(op, s_prev, batch2d, lin_w, lin_b):
    n = s_prev.shape[0]
    nblk = n // _BN
    nclass = lin_w.shape[1]

    def body(op_ref, s_ref, b_ref, w_ref, lb_ref, o_ref, sums, counts):
        i = pl.program_id(0)
        both = op_ref[0] + op_ref[1]
        osum = both[:, :_HID]
        dsum = both[:, _HID:_HID + 1]
        safe = jnp.where(dsum > 0.0, dsum, 1.0)
        h4 = jax.nn.relu(jnp.where(dsum > 0.0, osum / safe, 0.0) + s_ref[...])
        gids = lax.broadcasted_iota(jnp.int32, (_BN, _G), 1)
        oh = (b_ref[...] == gids).astype(jnp.float32)
        part_sums = lax.dot_general(oh, h4, (((0,), (0,)), ((), ())), precision=_PREC)
        part_counts = jnp.sum(oh, axis=0, keepdims=True)

        @pl.when(i == 0)
        def _():
            sums[...] = part_sums
            counts[...] = part_counts

        @pl.when(i > 0)
        def _():
            sums[...] = sums[...] + part_sums
            counts[...] = counts[...] + part_counts

        @pl.when(i == nblk - 1)
        def _():
            cnt = jnp.transpose(counts[...])
            pooled = sums[...] / jnp.maximum(cnt, 1.0)
            logits = jnp.dot(pooled, w_ref[...], precision=_PREC) + lb_ref[...]
            m = jnp.max(logits, axis=1, keepdims=True)
            z = logits - m
            o_ref[...] = z - jnp.log(jnp.sum(jnp.exp(z), axis=1, keepdims=True))

    return pl.pallas_call(
        body,
        grid=(nblk,),
        in_specs=[
            pl.BlockSpec((2, _BN, 128), lambda i: (0, i, 0)),
            pl.BlockSpec((_BN, _HID), lambda i: (i, 0)),
            pl.BlockSpec((_BN, 1), lambda i: (i, 0)),
            pl.BlockSpec((_HID, nclass), lambda i: (0, 0)),
            pl.BlockSpec((1, nclass), lambda i: (0, 0)),
        ],
        out_specs=[pl.BlockSpec((_G, nclass), lambda i: (0, 0))],
        out_shape=[jax.ShapeDtypeStruct((_G, nclass), jnp.float32)],
        scratch_shapes=[pltpu.VMEM((_G, _HID), jnp.float32),
                        pltpu.VMEM((1, _G), jnp.float32)],
    )(op, s_prev, batch2d, lin_w, lin_b)[0]


def _sc_mesh():
    return plsc.VectorSubcoreMesh(core_axis_name="c", subcore_axis_name="s")


_WPC = _CHUNKS * _WSC


def _sc_gather3(q, k, v, dst1d, src1d):
    tabs = (q, k, v)

    @functools.partial(
        pl.kernel,
        out_type=[jax.ShapeDtypeStruct((_EP, 128), jnp.float32)] * 3,
        mesh=_sc_mesh(),
        scratch_types=[
            pltpu.VMEM((_WPC,), jnp.int32),
            pltpu.VMEM((_WPC,), jnp.int32),
            pltpu.VMEM((_WSC, 128), jnp.float32),
            pltpu.VMEM((_WSC, 128), jnp.float32),
            pltpu.VMEM((_WSC, 128), jnp.float32),
            pltpu.VMEM((_WSC, 128), jnp.float32),
            pltpu.VMEM((_WSC, 128), jnp.float32),
            pltpu.VMEM((_WSC, 128), jnp.float32),
        ] + [pltpu.SemaphoreType.DMA] * 12,
    )
    def kern(q_hbm, k_hbm, v_hbm, dst_hbm, src_hbm, qd_hbm, ks_hbm, vs_hbm,
             di_v, si_v, q0, k0, v0, q1, k1, v1,
             g0a, g0b, g0c, g1a, g1b, g1c, w0a, w0b, w0c, w1a, w1b, w1c):
        wid = lax.axis_index("s") * 2 + lax.axis_index("c")
        wbase = pl.multiple_of(wid * _WPC, _WSC)
        pltpu.sync_copy(dst_hbm.at[pl.ds(wbase, _WPC)], di_v)
        pltpu.sync_copy(src_hbm.at[pl.ds(wbase, _WPC)], si_v)
        bufs = ((q0, k0, v0), (q1, k1, v1))
        gsems = ((g0a, g0b, g0c), (g1a, g1b, g1c))
        wsems = ((w0a, w0b, w0c), (w1a, w1b, w1c))
        hbms = (q_hbm, k_hbm, v_hbm)
        outs = (qd_hbm, ks_hbm, vs_hbm)

        @pl.loop(0, _CHUNKS // 2)
        def _(jj):
            for p in (0, 1):
                j = 2 * jj + p
                off = pl.multiple_of(j * _WSC, _WSC)
                base = pl.multiple_of(wbase + j * _WSC, _WSC)

                @pl.when(jj > 0)
                def _():
                    for t in range(3):
                        pltpu.make_async_copy(
                            bufs[p][t], outs[t].at[pl.ds(base, _WSC)],
                            wsems[p][t]).wait()

                gs = []
                for t in range(3):
                    idx = di_v if t == 0 else si_v
                    gs.append(pltpu.async_copy(
                        hbms[t].at[idx.at[pl.ds(off, _WSC)]], bufs[p][t],
                        gsems[p][t]))
                for t in range(3):
                    gs[t].wait()
                for t in range(3):
                    pltpu.async_copy(bufs[p][t], outs[t].at[pl.ds(base, _WSC)],
                                     wsems[p][t])

        for p in (0, 1):
            for t in range(3):
                pltpu.make_async_copy(
                    bufs[p][t], outs[t].at[pl.ds(wbase, _WSC)],
                    wsems[p][t]).wait()

    return kern(*tabs, dst1d, src1d)


def _sc_gather2(qkv, dst1d, src1d):
    @functools.partial(
        pl.kernel,
        out_type=[jax.ShapeDtypeStruct((_EP, 128), jnp.float32)] * 2,
        mesh=_sc_mesh(),
        scratch_types=[
            pltpu.VMEM((_WPC,), jnp.int32),
            pltpu.VMEM((_WPC,), jnp.int32),
            pltpu.VMEM((_WSC, 128), jnp.float32),
            pltpu.VMEM((_WSC, 128), jnp.float32),
            pltpu.VMEM((_WSC, 128), jnp.float32),
            pltpu.VMEM((_WSC, 128), jnp.float32),
        ] + [pltpu.SemaphoreType.DMA] * 8,
    )
    def kern(t_hbm, dst_hbm, src_hbm, dr_hbm, sr_hbm,
             di_v, si_v, d0, s0, d1, s1,
             g0a, g0b, g1a, g1b, w0a, w0b, w1a, w1b):
        wid = lax.axis_index("s") * 2 + lax.axis_index("c")
        wbase = pl.multiple_of(wid * _WPC, _WSC)
        pltpu.sync_copy(dst_hbm.at[pl.ds(wbase, _WPC)], di_v)
        pltpu.sync_copy(src_hbm.at[pl.ds(wbase, _WPC)], si_v)
        bufs = ((d0, s0), (d1, s1))
        gsems = ((g0a, g0b), (g1a, g1b))
        wsems = ((w0a, w0b), (w1a, w1b))
        idxs = (di_v, si_v)
        outs = (dr_hbm, sr_hbm)

        @pl.loop(0, _CHUNKS // 2)
        def _(jj):
            for p in (0, 1):
                j = 2 * jj + p
                off = pl.multiple_of(j * _WSC, _WSC)
                base = pl.multiple_of(wbase + j * _WSC, _WSC)

                @pl.when(jj > 0)
                def _():
                    for t in range(2):
                        pltpu.make_async_copy(
                            bufs[p][t], outs[t].at[pl.ds(base, _WSC)],
                            wsems[p][t]).wait()

                gs = [pltpu.async_copy(
                    t_hbm.at[idxs[t].at[pl.ds(off, _WSC)]], bufs[p][t],
                    gsems[p][t]) for t in range(2)]
                for t in range(2):
                    gs[t].wait()
                for t in range(2):
                    pltpu.async_copy(bufs[p][t], outs[t].at[pl.ds(base, _WSC)],
                                     wsems[p][t])

        for p in (0, 1):
            for t in range(2):
                pltpu.make_async_copy(
                    bufs[p][t], outs[t].at[pl.ds(wbase, _WSC)],
                    wsems[p][t]).wait()

    return kern(qkv, dst1d, src1d)


_NBUF = 2


def _scatter_phase(d_hbm, out_hbm, zd_hbm, acc, di_v, bufs, lsems, ssems,
                   c, s, wid):
    pltpu.sync_copy(zd_hbm, acc.at[pl.ds(s * _ROWS, _ROWS)])
    plsc.subcore_barrier()

    @pl.loop(0, _CHUNKS // _NBUF)
    def _(jj):
        lds = []
        for p in range(_NBUF):
            j = _NBUF * jj + p
            base = pl.multiple_of((wid * _CHUNKS + j) * _WSC, _WSC)

            @pl.when(jj > 0)
            def _():
                pltpu.make_async_copy(bufs[p], acc.at[di_v.at[j]],
                                      ssems[p]).wait()

            lds.append(pltpu.async_copy(d_hbm.at[pl.ds(base, _WSC)],
                                        bufs[p], lsems[p]))
        for p in range(_NBUF):
            j = _NBUF * jj + p
            lds[p].wait()
            pltpu.async_copy(bufs[p], acc.at[di_v.at[j]], ssems[p], add=True)

    for p in range(_NBUF):
        pltpu.make_async_copy(bufs[p], acc.at[di_v.at[0]], ssems[p]).wait()
    plsc.subcore_barrier()
    pltpu.sync_copy(acc.at[pl.ds(s * _ROWS, _ROWS)],
                    out_hbm.at[c, pl.ds(s * _ROWS, _ROWS)])
    plsc.subcore_barrier()


def _scatter_scratch():
    return [
        pltpu.VMEM_SHARED((_NP, 128), jnp.float32),
        pltpu.VMEM((_CHUNKS, _WSC), jnp.int32),
        pltpu.VMEM((_WSC, 128), jnp.float32),
        pltpu.VMEM((_WSC, 128), jnp.float32),
    ] + [pltpu.SemaphoreType.DMA] * (2 * _NBUF)


def _sc_scatter2(vw, ew, dst2d, zd):
    @functools.partial(
        pl.kernel,
        out_type=[jax.ShapeDtypeStruct((2, _NP, 128), jnp.float32)] * 2,
        mesh=_sc_mesh(),
        scratch_types=_scatter_scratch(),
    )
    def kern(vw_hbm, ew_hbm, dst_hbm, zd_hbm, op_hbm, dp_hbm, acc, di_v,
             b0, b1, l0, l1, s0, s1):
        c = lax.axis_index("c")
        s = lax.axis_index("s")
        wid = s * 2 + c
        pltpu.sync_copy(dst_hbm.at[pl.ds(wid * _CHUNKS, _CHUNKS)], di_v)
        for d_hbm, out_hbm in ((vw_hbm, op_hbm), (ew_hbm, dp_hbm)):
            _scatter_phase(d_hbm, out_hbm, zd_hbm, acc, di_v,
                           (b0, b1), (l0, l1), (s0, s1), c, s, wid)

    return kern(vw, ew, dst2d, zd)


def _sc_scatter1(data, dst2d, zd):
    @functools.partial(
        pl.kernel,
        out_type=jax.ShapeDtypeStruct((2, _NP, 128), jnp.float32),
        mesh=_sc_mesh(),
        scratch_types=_scatter_scratch(),
    )
    def kern(d_hbm, dst_hbm, zd_hbm, op_hbm, acc, di_v,
             b0, b1, l0, l1, s0, s1):
        c = lax.axis_index("c")
        s = lax.axis_index("s")
        wid = s * 2 + c
        pltpu.sync_copy(dst_hbm.at[pl.ds(wid * _CHUNKS, _CHUNKS)], di_v)
        _scatter_phase(d_hbm, op_hbm, zd_hbm, acc, di_v,
                       (b0, b1), (l0, l1), (s0, s1), c, s, wid)

    return kern(data, dst2d, zd)


def kernel(x, edge_index, batch, params):
    pad = jnp.zeros((_EP - _E,), jnp.int32)
    src1d = jnp.concatenate([edge_index[0], pad])
    dst1d = jnp.concatenate([edge_index[1], pad])
    dst2d = dst1d.reshape(_EP // _WSC, _WSC)
    batch2d = batch.reshape(_N, 1)
    z128 = jnp.zeros((_ROWS, 128), jnp.float32)

    op = dp = s_prev = None
    for li, nh in enumerate(_HEADS):
        p = params["layers"][li]
        args = (p["Wq"], p["bq"].reshape(1, -1), p["Wk"], p["bk"].reshape(1, -1),
                p["Wv"], p["bv"].reshape(1, -1), p["Ws"], p["bs"].reshape(1, -1))
        if li == 0:
            q, k, v, s = _tc_qkvs_first(x, *args)
            qd, ks, vs = _sc_gather3(q, k, v, dst1d, src1d)
            alpha, gmax = _tc_alpha(qd, ks, nh, 0, 0)
            ew, vw = _tc_weight(alpha, gmax, vs, nh, 0)
        elif nh == 8:
            q, k, v, s = _tc_qkvs_next(op, dp, s_prev, *args)
            qd, ks, vs = _sc_gather3(q, k, v, dst1d, src1d)
            alpha, gmax = _tc_alpha(qd, ks, nh, 0, 0)
            ew, vw = _tc_weight(alpha, gmax, vs, nh, 0)
        else:
            qkv, s = _tc_qkvs_last(op, dp, s_prev, *args)
            drows, srows = _sc_gather2(qkv, dst1d, src1d)
            alpha, gmax = _tc_alpha(drows, srows, nh, 0, _HID)
            cw = _tc_weight1(alpha, gmax, srows, 2 * _HID)
            op = _sc_scatter1(cw, dst2d, z128)
            s_prev = s
            break
        op, dp = _sc_scatter2(vw, ew, dst2d, z128)
        s_prev = s

    return _tc_pool(op, s_prev, batch2d,
                    params["lin_W"], params["lin_b"].reshape(1, -1))


# speedup vs baseline: 1.8248x; 1.8248x over previous
import functools

import numpy as np
import jax
import jax.numpy as jnp
from jax import lax
from jax.experimental import pallas as pl
from jax.experimental.pallas import tpu as pltpu
from jax.experimental.pallas import tpu_sc as plsc

_N = 10000
_NP = 10112
_E = 320000
_EP = 327680
_HID = 16
_HEADS = (8, 8, 8, 1)
_G = 64
_PREC = lax.Precision.HIGHEST

_BN = 2000
_BE = 2048
_WSC = 128
_NWORK = 32
_CHUNKS = _EP // (_NWORK * _WSC)
_ROWS = _NP // 16


def _hrep(nh, d):
    rows = lax.broadcasted_iota(jnp.int32, (nh, d), 0)
    cols = lax.broadcasted_iota(jnp.int32, (nh, d), 1)
    return (cols // (d // nh) == rows).astype(jnp.float32)


def _combine_prev(op_blk, dp_blk, s_blk):
    osum = op_blk[0] + op_blk[1]
    dsum = (dp_blk[0] + dp_blk[1])[:, :8]
    drep = jnp.dot(dsum, _hrep(8, 128), precision=_PREC)
    safe = jnp.where(drep > 0.0, drep, 1.0)
    return jax.nn.relu(jnp.where(drep > 0.0, osum / safe, 0.0) + s_blk)


def _weight_specs(din, dout):
    w = pl.BlockSpec((din, dout), lambda i: (0, 0))
    b = pl.BlockSpec((1, dout), lambda i: (0, 0))
    return [w, b, w, b, w, b, w, b]


def _proj(hb, w_ref, b_ref):
    return jnp.dot(hb, w_ref[...], precision=_PREC) + b_ref[...]


def _tc_qkvs_first(h, wq, bq, wk, bk, wv, bv, ws, bs):
    n, din = h.shape
    dout = wq.shape[1]

    def body(h_ref, wq_r, bq_r, wk_r, bk_r, wv_r, bv_r, ws_r, bs_r,
             q_o, k_o, v_o, s_o):
        hb = h_ref[...]
        q_o[...] = _proj(hb, wq_r, bq_r)
        k_o[...] = _proj(hb, wk_r, bk_r)
        v_o[...] = _proj(hb, wv_r, bv_r)
        s_o[...] = _proj(hb, ws_r, bs_r)

    return pl.pallas_call(
        body,
        grid=(n // _BN,),
        in_specs=[pl.BlockSpec((_BN, din), lambda i: (i, 0))] + _weight_specs(din, dout),
        out_specs=[pl.BlockSpec((_BN, dout), lambda i: (i, 0))] * 4,
        out_shape=[jax.ShapeDtypeStruct((n, dout), jnp.float32)] * 4,
    )(h, wq, bq, wk, bk, wv, bv, ws, bs)


def _tc_qkvs_next(op, dp, s_prev, wq, bq, wk, bk, wv, bv, ws, bs):
    n = s_prev.shape[0]
    din = s_prev.shape[1]
    dout = wq.shape[1]

    def body(op_ref, dp_ref, s_ref, wq_r, bq_r, wk_r, bk_r, wv_r, bv_r, ws_r, bs_r,
             q_o, k_o, v_o, s_o):
        hb = _combine_prev(op_ref[...], dp_ref[...], s_ref[...])
        q_o[...] = _proj(hb, wq_r, bq_r)
        k_o[...] = _proj(hb, wk_r, bk_r)
        v_o[...] = _proj(hb, wv_r, bv_r)
        s_o[...] = _proj(hb, ws_r, bs_r)

    return pl.pallas_call(
        body,
        grid=(n // _BN,),
        in_specs=[
            pl.BlockSpec((2, _BN, din), lambda i: (0, i, 0)),
            pl.BlockSpec((2, _BN, 128), lambda i: (0, i, 0)),
            pl.BlockSpec((_BN, din), lambda i: (i, 0)),
        ] + _weight_specs(din, dout),
        out_specs=[pl.BlockSpec((_BN, dout), lambda i: (i, 0))] * 4,
        out_shape=[jax.ShapeDtypeStruct((n, dout), jnp.float32)] * 4,
    )(op, dp, s_prev, wq, bq, wk, bk, wv, bv, ws, bs)


def _tc_qkvs_last(op, dp, s_prev, wq, bq, wk, bk, wv, bv, ws, bs):
    n = s_prev.shape[0]
    din = s_prev.shape[1]
    dout = wq.shape[1]

    def body(op_ref, dp_ref, s_ref, wq_r, bq_r, wk_r, bk_r, wv_r, bv_r, ws_r, bs_r,
             qkv_o, s_o):
        hb = _combine_prev(op_ref[...], dp_ref[...], s_ref[...])
        q = _proj(hb, wq_r, bq_r)
        k = _proj(hb, wk_r, bk_r)
        v = _proj(hb, wv_r, bv_r)
        z = jnp.zeros((q.shape[0], 128 - 3 * dout), jnp.float32)
        qkv_o[...] = jnp.concatenate([q, k, v, z], axis=1)
        s_o[...] = _proj(hb, ws_r, bs_r)

    return pl.pallas_call(
        body,
        grid=(n // _BN,),
        in_specs=[
            pl.BlockSpec((2, _BN, din), lambda i: (0, i, 0)),
            pl.BlockSpec((2, _BN, 128), lambda i: (0, i, 0)),
            pl.BlockSpec((_BN, din), lambda i: (i, 0)),
        ] + _weight_specs(din, dout),
        out_specs=[pl.BlockSpec((_BN, 128), lambda i: (i, 0)),
                   pl.BlockSpec((_BN, dout), lambda i: (i, 0))],
        out_shape=[jax.ShapeDtypeStruct((n, 128), jnp.float32),
                   jax.ShapeDtypeStruct((n, dout), jnp.float32)],
    )(op, dp, s_prev, wq, bq, wk, bk, wv, bv, ws, bs)


def _tc_alpha(qrows, krows, nh, qoff, koff):
    e = qrows.shape[0]
    dc = nh * _HID
    nblk = e // _BE

    def body(qd_ref, ks_ref, a_ref, m_ref):
        i = pl.program_id(0)
        prod = qd_ref[:, qoff:qoff + dc] * ks_ref[:, koff:koff + dc]
        rt = _hrep(nh, dc).T
        a = jnp.dot(prod, rt, precision=_PREC) * (1.0 / np.sqrt(_HID))
        if nh < 8:
            a = jnp.concatenate([a, jnp.zeros((a.shape[0], 8 - nh), jnp.float32)], axis=1)
        a_ref[...] = a
        bm = jnp.max(a, axis=0, keepdims=True)

        @pl.when(i == 0)
        def _():
            m_ref[...] = bm

        @pl.when(i > 0)
        def _():
            m_ref[...] = jnp.maximum(m_ref[...], bm)

    return pl.pallas_call(
        body,
        grid=(nblk,),
        in_specs=[pl.BlockSpec((_BE, 128), lambda i: (i, 0))] * 2,
        out_specs=[pl.BlockSpec((_BE, 8), lambda i: (i, 0)),
                   pl.BlockSpec((1, 8), lambda i: (0, 0))],
        out_shape=[jax.ShapeDtypeStruct((e, 8), jnp.float32),
                   jax.ShapeDtypeStruct((1, 8), jnp.float32)],
    )(qrows, krows)


def _tc_weight(alpha, gmax, vrows, nh, voff):
    e = alpha.shape[0]
    dc = nh * _HID
    nblk = e // _BE

    def body(a_ref, m_ref, vs_ref, ew_ref, vw_ref):
        i = pl.program_id(0)
        rows = lax.broadcasted_iota(jnp.int32, (_BE, 1), 0) + i * _BE
        valid = (rows < _E).astype(jnp.float32)
        e8 = jnp.exp(a_ref[...] - m_ref[...]) * valid
        ew_ref[...] = jnp.concatenate(
            [e8, jnp.zeros((e8.shape[0], 120), jnp.float32)], axis=1)
        vs = vs_ref[:, voff:voff + dc]
        vw_ref[...] = vs * jnp.dot(e8[:, :nh], _hrep(nh, dc), precision=_PREC)

    return pl.pallas_call(
        body,
        grid=(nblk,),
        in_specs=[pl.BlockSpec((_BE, 8), lambda i: (i, 0)),
                  pl.BlockSpec((1, 8), lambda i: (0, 0)),
                  pl.BlockSpec((_BE, 128), lambda i: (i, 0))],
        out_specs=[pl.BlockSpec((_BE, 128), lambda i: (i, 0)),
                   pl.BlockSpec((_BE, 128), lambda i: (i, 0))],
        out_shape=[jax.ShapeDtypeStruct((e, 128), jnp.float32),
                   jax.ShapeDtypeStruct((e, 128), jnp.float32)],
    )(alpha, gmax, vrows)


def _tc_weight1(alpha, gmax, vrows, voff):
    e = alpha.shape[0]
    nblk = e // _BE

    def body(a_ref, m_ref, vs_ref, cw_ref):
        i = pl.program_id(0)
        rows = lax.broadcasted_iota(jnp.int32, (_BE, 1), 0) + i * _BE
        valid = (rows < _E).astype(jnp.float32)
        e8 = jnp.exp(a_ref[...] - m_ref[...]) * valid
        vs = vs_ref[:, voff:voff + _HID]
        cw_ref[...] = jnp.concatenate(
            [vs * e8[:, 0:1], e8,
             jnp.zeros((e8.shape[0], 128 - _HID - 8), jnp.float32)], axis=1)

    return pl.pallas_call(
        body,
        grid=(nblk,),
        in_specs=[pl.BlockSpec((_BE, 8), lambda i: (i, 0)),
                  pl.BlockSpec((1, 8), lambda i: (0, 0)),
                  pl.BlockSpec((_BE, 128), lambda i: (i, 0))],
        out_specs=[pl.BlockSpec((_BE, 128), lambda i: (i, 0))],
        out_shape=[jax.ShapeDtypeStruct((e, 128), jnp.float32)],
    )(alpha, gmax, vrows)[0]


def _tc_pool(op, s_prev, batch2d, lin_w, lin_b):
    n = s_prev.shape[0]
    nblk = n // _BN
    nclass = lin_w.shape[1]

    def body(op_ref, s_ref, b_ref, w_ref, lb_ref, o_ref, sums, counts):
        i = pl.program_id(0)
        both = op_ref[0] + op_ref[1]
        osum = both[:, :_HID]
        dsum = both[:, _HID:_HID + 1]
        safe = jnp.where(dsum > 0.0, dsum, 1.0)
        h4 = jax.nn.relu(jnp.where(dsum > 0.0, osum / safe, 0.0) + s_ref[...])
        gids = lax.broadcasted_iota(jnp.int32, (_BN, _G), 1)
        oh = (b_ref[...] == gids).astype(jnp.float32)
        part_sums = lax.dot_general(oh, h4, (((0,), (0,)), ((), ())), precision=_PREC)
        part_counts = jnp.sum(oh, axis=0, keepdims=True)

        @pl.when(i == 0)
        def _():
            sums[...] = part_sums
            counts[...] = part_counts

        @pl.when(i > 0)
        def _():
            sums[...] = sums[...] + part_sums
            counts[...] = counts[...] + part_counts

        @pl.when(i == nblk - 1)
        def _():
            cnt = jnp.transpose(counts[...])
            pooled = sums[...] / jnp.maximum(cnt, 1.0)
            logits = jnp.dot(pooled, w_ref[...], precision=_PREC) + lb_ref[...]
            m = jnp.max(logits, axis=1, keepdims=True)
            z = logits - m
            o_ref[...] = z - jnp.log(jnp.sum(jnp.exp(z), axis=1, keepdims=True))

    return pl.pallas_call(
        body,
        grid=(nblk,),
        in_specs=[
            pl.BlockSpec((2, _BN, 128), lambda i: (0, i, 0)),
            pl.BlockSpec((_BN, _HID), lambda i: (i, 0)),
            pl.BlockSpec((_BN, 1), lambda i: (i, 0)),
            pl.BlockSpec((_HID, nclass), lambda i: (0, 0)),
            pl.BlockSpec((1, nclass), lambda i: (0, 0)),
        ],
        out_specs=[pl.BlockSpec((_G, nclass), lambda i: (0, 0))],
        out_shape=[jax.ShapeDtypeStruct((_G, nclass), jnp.float32)],
        scratch_shapes=[pltpu.VMEM((_G, _HID), jnp.float32),
                        pltpu.VMEM((1, _G), jnp.float32)],
    )(op, s_prev, batch2d, lin_w, lin_b)[0]


def _sc_mesh():
    return plsc.VectorSubcoreMesh(core_axis_name="c", subcore_axis_name="s")


_WPC = _CHUNKS * _WSC
_STG = 632


def _stage_table(t_hbm, tab, s):
    @pl.when(s < 15)
    def _():
        off = pl.multiple_of(s * _STG, 8)
        pltpu.sync_copy(t_hbm.at[pl.ds(off, _STG)], tab.at[pl.ds(off, _STG)])

    @pl.when(s == 15)
    def _():
        pltpu.sync_copy(t_hbm.at[pl.ds(15 * _STG, _N - 15 * _STG)],
                        tab.at[pl.ds(15 * _STG, _N - 15 * _STG)])


def _gather_epoch(tab, idx_hbm, out_hbm, dis, bufs, gsems, wsems, wid):
    @pl.loop(0, _CHUNKS // 2)
    def _(jj):
        gs = [None, None]
        for p in (0, 1):
            j = 2 * jj + p
            base = pl.multiple_of(wid * _WPC + j * _WSC, _WSC)

            @pl.when(jj > 0)
            def _():
                pltpu.make_async_copy(bufs[p], out_hbm.at[pl.ds(base, _WSC)],
                                      wsems[p]).wait()

            pltpu.sync_copy(idx_hbm.at[pl.ds(base, _WSC)], dis[p])
            gs[p] = pltpu.async_copy(tab.at[dis[p]], bufs[p], gsems[p])
        for p in (0, 1):
            j = 2 * jj + p
            base = pl.multiple_of(wid * _WPC + j * _WSC, _WSC)
            gs[p].wait()
            pltpu.async_copy(bufs[p], out_hbm.at[pl.ds(base, _WSC)], wsems[p])

    for p in (0, 1):
        pltpu.make_async_copy(bufs[p], out_hbm.at[pl.ds(wid * _WPC, _WSC)],
                              wsems[p]).wait()


def _gather_scratch():
    return [
        pltpu.VMEM_SHARED((_N, 128), jnp.float32),
        pltpu.VMEM((_WSC,), jnp.int32),
        pltpu.VMEM((_WSC,), jnp.int32),
        pltpu.VMEM((_WSC, 128), jnp.float32),
        pltpu.VMEM((_WSC, 128), jnp.float32),
    ] + [pltpu.SemaphoreType.DMA] * 4


def _sc_gather3(q, k, v, dst1d, src1d):
    @functools.partial(
        pl.kernel,
        out_type=[jax.ShapeDtypeStruct((_EP, 128), jnp.float32)] * 3,
        mesh=_sc_mesh(),
        scratch_types=_gather_scratch(),
    )
    def kern(q_hbm, k_hbm, v_hbm, dst_hbm, src_hbm, qd_hbm, ks_hbm, vs_hbm,
             tab, di0, di1, b0, b1, g0, g1, w0, w1):
        s = lax.axis_index("s")
        wid = s * 2 + lax.axis_index("c")
        for t_hbm, idx_hbm, out_hbm in ((q_hbm, dst_hbm, qd_hbm),
                                        (k_hbm, src_hbm, ks_hbm),
                                        (v_hbm, src_hbm, vs_hbm)):
            _stage_table(t_hbm, tab, s)
            plsc.subcore_barrier()
            _gather_epoch(tab, idx_hbm, out_hbm, (di0, di1), (b0, b1),
                          (g0, g1), (w0, w1), wid)
            plsc.subcore_barrier()

    return kern(q, k, v, dst1d, src1d)


def _sc_gather2(qkv, dst1d, src1d):
    @functools.partial(
        pl.kernel,
        out_type=[jax.ShapeDtypeStruct((_EP, 128), jnp.float32)] * 2,
        mesh=_sc_mesh(),
        scratch_types=_gather_scratch(),
    )
    def kern(t_hbm, dst_hbm, src_hbm, dr_hbm, sr_hbm,
             tab, di0, di1, b0, b1, g0, g1, w0, w1):
        s = lax.axis_index("s")
        wid = s * 2 + lax.axis_index("c")
        _stage_table(t_hbm, tab, s)
        plsc.subcore_barrier()
        for idx_hbm, out_hbm in ((dst_hbm, dr_hbm), (src_hbm, sr_hbm)):
            _gather_epoch(tab, idx_hbm, out_hbm, (di0, di1), (b0, b1),
                          (g0, g1), (w0, w1), wid)
        plsc.subcore_barrier()

    return kern(qkv, dst1d, src1d)


_NBUF = 2


def _scatter_phase(d_hbm, out_hbm, zd_hbm, acc, di_v, bufs, lsems, ssems,
                   c, s, wid):
    pltpu.sync_copy(zd_hbm, acc.at[pl.ds(s * _ROWS, _ROWS)])
    plsc.subcore_barrier()

    @pl.loop(0, _CHUNKS // _NBUF)
    def _(jj):
        lds = []
        for p in range(_NBUF):
            j = _NBUF * jj + p
            base = pl.multiple_of((wid * _CHUNKS + j) * _WSC, _WSC)

            @pl.when(jj > 0)
            def _():
                pltpu.make_async_copy(bufs[p], acc.at[di_v.at[j]],
                                      ssems[p]).wait()

            lds.append(pltpu.async_copy(d_hbm.at[pl.ds(base, _WSC)],
                                        bufs[p], lsems[p]))
        for p in range(_NBUF):
            j = _NBUF * jj + p
            lds[p].wait()
            pltpu.async_copy(bufs[p], acc.at[di_v.at[j]], ssems[p], add=True)

    for p in range(_NBUF):
        pltpu.make_async_copy(bufs[p], acc.at[di_v.at[0]], ssems[p]).wait()
    plsc.subcore_barrier()
    pltpu.sync_copy(acc.at[pl.ds(s * _ROWS, _ROWS)],
                    out_hbm.at[c, pl.ds(s * _ROWS, _ROWS)])
    plsc.subcore_barrier()


def _scatter_scratch():
    return [
        pltpu.VMEM_SHARED((_NP, 128), jnp.float32),
        pltpu.VMEM((_CHUNKS, _WSC), jnp.int32),
        pltpu.VMEM((_WSC, 128), jnp.float32),
        pltpu.VMEM((_WSC, 128), jnp.float32),
    ] + [pltpu.SemaphoreType.DMA] * (2 * _NBUF)


def _sc_scatter2(vw, ew, dst2d, zd):
    @functools.partial(
        pl.kernel,
        out_type=[jax.ShapeDtypeStruct((2, _NP, 128), jnp.float32)] * 2,
        mesh=_sc_mesh(),
        scratch_types=_scatter_scratch(),
    )
    def kern(vw_hbm, ew_hbm, dst_hbm, zd_hbm, op_hbm, dp_hbm, acc, di_v,
             b0, b1, l0, l1, s0, s1):
        c = lax.axis_index("c")
        s = lax.axis_index("s")
        wid = s * 2 + c
        pltpu.sync_copy(dst_hbm.at[pl.ds(wid * _CHUNKS, _CHUNKS)], di_v)
        for d_hbm, out_hbm in ((vw_hbm, op_hbm), (ew_hbm, dp_hbm)):
            _scatter_phase(d_hbm, out_hbm, zd_hbm, acc, di_v,
                           (b0, b1), (l0, l1), (s0, s1), c, s, wid)

    return kern(vw, ew, dst2d, zd)


def _sc_scatter1(data, dst2d, zd):
    @functools.partial(
        pl.kernel,
        out_type=jax.ShapeDtypeStruct((2, _NP, 128), jnp.float32),
        mesh=_sc_mesh(),
        scratch_types=_scatter_scratch(),
    )
    def kern(d_hbm, dst_hbm, zd_hbm, op_hbm, acc, di_v,
             b0, b1, l0, l1, s0, s1):
        c = lax.axis_index("c")
        s = lax.axis_index("s")
        wid = s * 2 + c
        pltpu.sync_copy(dst_hbm.at[pl.ds(wid * _CHUNKS, _CHUNKS)], di_v)
        _scatter_phase(d_hbm, op_hbm, zd_hbm, acc, di_v,
                       (b0, b1), (l0, l1), (s0, s1), c, s, wid)

    return kern(data, dst2d, zd)


def kernel(x, edge_index, batch, params):
    pad = jnp.zeros((_EP - _E,), jnp.int32)
    src1d = jnp.concatenate([edge_index[0], pad])
    dst1d = jnp.concatenate([edge_index[1], pad])
    dst2d = dst1d.reshape(_EP // _WSC, _WSC)
    batch2d = batch.reshape(_N, 1)
    z128 = jnp.zeros((_ROWS, 128), jnp.float32)

    op = dp = s_prev = None
    for li, nh in enumerate(_HEADS):
        p = params["layers"][li]
        args = (p["Wq"], p["bq"].reshape(1, -1), p["Wk"], p["bk"].reshape(1, -1),
                p["Wv"], p["bv"].reshape(1, -1), p["Ws"], p["bs"].reshape(1, -1))
        if li == 0:
            q, k, v, s = _tc_qkvs_first(x, *args)
            qd, ks, vs = _sc_gather3(q, k, v, dst1d, src1d)
            alpha, gmax = _tc_alpha(qd, ks, nh, 0, 0)
            ew, vw = _tc_weight(alpha, gmax, vs, nh, 0)
        elif nh == 8:
            q, k, v, s = _tc_qkvs_next(op, dp, s_prev, *args)
            qd, ks, vs = _sc_gather3(q, k, v, dst1d, src1d)
            alpha, gmax = _tc_alpha(qd, ks, nh, 0, 0)
            ew, vw = _tc_weight(alpha, gmax, vs, nh, 0)
        else:
            qkv, s = _tc_qkvs_last(op, dp, s_prev, *args)
            drows, srows = _sc_gather2(qkv, dst1d, src1d)
            alpha, gmax = _tc_alpha(drows, srows, nh, 0, _HID)
            cw = _tc_weight1(alpha, gmax, srows, 2 * _HID)
            op = _sc_scatter1(cw, dst2d, z128)
            s_prev = s
            break
        op, dp = _sc_scatter2(vw, ew, dst2d, z128)
        s_prev = s

    return _tc_pool(op, s_prev, batch2d,
                    params["lin_W"], params["lin_b"].reshape(1, -1))
